# Initial kernel scaffold; baseline (speedup 1.0000x reference)
import jax
import jax.numpy as jnp
from jax.experimental import pallas as pl


def kernel(input_x, edge_index, W1, a_src1, a_dst1, b1, W2, a_src2, a_dst2, b2):
    raise NotImplementedError("write your pallas kernel here")


# speedup vs baseline: 11.9077x; 11.9077x over previous
import functools

import jax
import jax.numpy as jnp
from jax import lax
from jax.experimental import pallas as pl
from jax.experimental.pallas import tpu as pltpu
from jax.experimental.pallas import tpu_sc as plsc

N = 10000
E = 320000
D = 128

NC = 2
NS = 16
L = 16

CHUNK = 64
NCHUNK = 160
IDXBLK = 8
NBLK = NCHUNK // IDXBLK
EPW = CHUNK * NCHUNK
EP = EPW * NC * NS
N_ACC = 10112
ROWS_PER_TILE = N_ACC // NS
DEN_ROWS = 80

_f32 = jnp.float32
_i32 = jnp.int32


def _edge_body(src_hbm, dst_hbm, h_hbm, as_hbm, ad_hbm, zb_hbm,
               num_out, den_out,
               src_idx, dst_idx, as_t, ad_t, rows, den_t, sem,
               num_acc):
    cid = lax.axis_index("c")
    sid = lax.axis_index("s")
    wid = cid * NS + sid

    r0 = sid * ROWS_PER_TILE
    pltpu.sync_copy(zb_hbm.at[pl.ds(0, CHUNK)], rows)
    pltpu.sync_copy(zb_hbm.at[pl.ds(0, DEN_ROWS)], den_t)
    for t in range(ROWS_PER_TILE // CHUNK + 1):
        sz = min(CHUNK, ROWS_PER_TILE - t * CHUNK)
        if sz <= 0:
            break
        pltpu.sync_copy(rows.at[pl.ds(0, sz)],
                        num_acc.at[pl.ds(r0 + t * CHUNK, sz)])

    pltpu.sync_copy(as_hbm, as_t)
    pltpu.sync_copy(ad_hbm, ad_t)

    plsc.subcore_barrier()

    ebase = wid * EPW
    lane = lax.iota(_i32, L)

    def chunk_body(j, jj):
        cp = pltpu.async_copy(h_hbm.at[src_idx.at[jj]], rows, sem)

        ws = []
        ds16 = []
        for k in range(CHUNK // L):
            s16 = src_idx[jj, pl.ds(k * L, L)]
            d16 = dst_idx[jj, pl.ds(k * L, L)]
            e = plsc.load_gather(as_t, [s16]) + plsc.load_gather(ad_t, [d16])
            e = jnp.where(e >= 0.0, e, e * jnp.float32(0.2))
            w = jnp.exp(e)
            gid = ebase + j * CHUNK + k * L + lane
            w = jnp.where(gid < E, w, jnp.float32(0.0))
            ws.append(w)
            ds16.append(d16)

        cp.wait()

        for k in range(CHUNK // L):
            for t in range(L):
                i = k * L + t
                sel = lane == t
                m = jnp.sum(jnp.where(sel, ws[k], jnp.float32(0.0)))
                di = jnp.sum(jnp.where(sel, ds16[k], 0))
                for g in range(D // L):
                    sl = pl.ds(g * L, L)
                    rows[i, sl] = rows[i, sl] * m
                dr = lax.shift_right_logical(di, 7)
                dca = lax.bitwise_and(di, 112)
                tin = lax.bitwise_and(di, 15)
                dsl = pl.ds(dca, L)
                den_t[dr, dsl] = den_t[dr, dsl] + jnp.where(
                    lane == tin, m, jnp.float32(0.0))

        pltpu.sync_copy(rows, num_acc.at[dst_idx.at[jj]], add=True)

    def block_body(b, carry):
        pltpu.sync_copy(src_hbm.at[wid, pl.ds(b * IDXBLK, IDXBLK)], src_idx)
        pltpu.sync_copy(dst_hbm.at[wid, pl.ds(b * IDXBLK, IDXBLK)], dst_idx)

        def inner(jj, c):
            chunk_body(b * IDXBLK + jj, jj)
            return c

        return lax.fori_loop(0, IDXBLK, inner, carry)

    lax.fori_loop(0, NBLK, block_body, 0)

    pltpu.sync_copy(den_t, den_out.at[wid])

    plsc.subcore_barrier()

    for t in range(ROWS_PER_TILE // CHUNK + 1):
        sz = min(CHUNK, ROWS_PER_TILE - t * CHUNK)
        if sz <= 0:
            break
        r = r0 + t * CHUNK
        pltpu.sync_copy(num_acc.at[pl.ds(r, sz)], rows.at[pl.ds(0, sz)])
        pltpu.sync_copy(rows.at[pl.ds(0, sz)],
                        num_out.at[cid, pl.ds(r, sz)])


_edge_kernel = functools.partial(
    pl.kernel,
    out_type=[
        jax.ShapeDtypeStruct((NC, N_ACC, D), _f32),
        jax.ShapeDtypeStruct((NC * NS, DEN_ROWS, D), _f32),
    ],
    mesh=plsc.VectorSubcoreMesh(core_axis_name="c", subcore_axis_name="s",
                                num_cores=NC, num_subcores=NS),
    compiler_params=pltpu.CompilerParams(needs_layout_passes=False),
    scratch_types=[
        pltpu.VMEM((IDXBLK, CHUNK), _i32),
        pltpu.VMEM((IDXBLK, CHUNK), _i32),
        pltpu.VMEM((N,), _f32),
        pltpu.VMEM((N,), _f32),
        pltpu.VMEM((CHUNK, D), _f32),
        pltpu.VMEM((DEN_ROWS, D), _f32),
        pltpu.SemaphoreType.DMA,
        pltpu.VMEM_SHARED((N_ACC, D), _f32),
    ],
)(_edge_body)


BN = 400
GRID = N // BN


def _mm_body(x_ref, w_ref, a_ref, h_ref, asad_ref):
    h = jnp.dot(x_ref[...], w_ref[...], preferred_element_type=_f32)
    h_ref[...] = h
    asad_ref[...] = jnp.dot(h, a_ref[...], preferred_element_type=_f32)


def _matmul(x, w, a):
    return pl.pallas_call(
        _mm_body,
        grid=(GRID,),
        in_specs=[
            pl.BlockSpec((BN, D), lambda i: (i, 0)),
            pl.BlockSpec((D, D), lambda i: (0, 0)),
            pl.BlockSpec((D, D), lambda i: (0, 0)),
        ],
        out_specs=[
            pl.BlockSpec((BN, D), lambda i: (i, 0)),
            pl.BlockSpec((BN, D), lambda i: (i, 0)),
        ],
        out_shape=[
            jax.ShapeDtypeStruct((N, D), _f32),
            jax.ShapeDtypeStruct((N, D), _f32),
        ],
    )(x, w, a)


def _combine_mm_body(num_ref, den_ref, b_ref, w_ref, a_ref, h_ref, asad_ref):
    x2 = jnp.maximum(
        num_ref[...] / (den_ref[...] + jnp.float32(1e-16)) + b_ref[...], 0.0)
    h = jnp.dot(x2, w_ref[...], preferred_element_type=_f32)
    h_ref[...] = h
    asad_ref[...] = jnp.dot(h, a_ref[...], preferred_element_type=_f32)


def _combine_mm(num, den, b, w, a):
    return pl.pallas_call(
        _combine_mm_body,
        grid=(GRID,),
        in_specs=[
            pl.BlockSpec((BN, D), lambda i: (i, 0)),
            pl.BlockSpec((BN, 1), lambda i: (i, 0)),
            pl.BlockSpec((1, D), lambda i: (0, 0)),
            pl.BlockSpec((D, D), lambda i: (0, 0)),
            pl.BlockSpec((D, D), lambda i: (0, 0)),
        ],
        out_specs=[
            pl.BlockSpec((BN, D), lambda i: (i, 0)),
            pl.BlockSpec((BN, D), lambda i: (i, 0)),
        ],
        out_shape=[
            jax.ShapeDtypeStruct((N, D), _f32),
            jax.ShapeDtypeStruct((N, D), _f32),
        ],
    )(num, den, b, w, a)


def _combine_body(num_ref, den_ref, b_ref, out_ref):
    out_ref[...] = jnp.maximum(
        num_ref[...] / (den_ref[...] + jnp.float32(1e-16)) + b_ref[...], 0.0)


def _combine(num, den, b):
    return pl.pallas_call(
        _combine_body,
        grid=(GRID,),
        in_specs=[
            pl.BlockSpec((BN, D), lambda i: (i, 0)),
            pl.BlockSpec((BN, 1), lambda i: (i, 0)),
            pl.BlockSpec((1, D), lambda i: (0, 0)),
        ],
        out_specs=pl.BlockSpec((BN, D), lambda i: (i, 0)),
        out_shape=jax.ShapeDtypeStruct((N, D), _f32),
    )(num, den, b)


def kernel(input_x, edge_index, W1, a_src1, a_dst1, b1, W2, a_src2, a_dst2, b2):
    ei = jnp.pad(edge_index, ((0, 0), (0, EP - E)))
    src_r = ei[0].reshape(NC * NS, NCHUNK, CHUNK)
    dst_r = ei[1].reshape(NC * NS, NCHUNK, CHUNK)

    zb = jnp.zeros((N_ACC, D), _f32)

    A1 = jnp.zeros((D, D), _f32).at[:, 0].set(a_src1).at[:, 1].set(a_dst1)
    A2 = jnp.zeros((D, D), _f32).at[:, 0].set(a_src2).at[:, 1].set(a_dst2)

    def _layer_edges(h, as_, ad_):
        nump, denp = _edge_kernel(src_r, dst_r, h, as_, ad_, zb)
        num = (nump[0] + nump[1])[:N]
        den = denp.reshape(NC * NS, DEN_ROWS * D).sum(axis=0)[:N]
        return num, den.reshape(N, 1)

    h1, asad1 = _matmul(input_x, W1, A1)
    num1, den1 = _layer_edges(h1, asad1[:, 0], asad1[:, 1])
    h2, asad2 = _combine_mm(num1, den1, b1.reshape(1, D), W2, A2)
    num2, den2 = _layer_edges(h2, asad2[:, 0], asad2[:, 1])
    return _combine(num2, den2, b2.reshape(1, D))


# speedup vs baseline: 14.7492x; 1.2386x over previous
import functools

import jax
import jax.numpy as jnp
from jax import lax
from jax.experimental import pallas as pl
from jax.experimental.pallas import tpu as pltpu
from jax.experimental.pallas import tpu_sc as plsc

N = 10000
E = 320000
D = 128

NC = 2
NS = 16
L = 16

CHUNK = 64
NCHUNK = 160
IDXBLK = 8
NBLK = NCHUNK // IDXBLK
EPW = CHUNK * NCHUNK
EP = EPW * NC * NS
N_ACC = 10112
ROWS_PER_TILE = N_ACC // NS
DEN_ROWS = 80

_f32 = jnp.float32
_i32 = jnp.int32


def _edge_body(src_hbm, dst_hbm, h_hbm, as_hbm, ad_hbm, zb_hbm,
               num_out, den_out,
               src_idx, dst_idx, as_t, ad_t, rows, rows2, den_t, sem, sem2,
               num_acc):
    cid = lax.axis_index("c")
    sid = lax.axis_index("s")
    wid = cid * NS + sid

    r0 = sid * ROWS_PER_TILE
    pltpu.sync_copy(zb_hbm.at[pl.ds(0, CHUNK)], rows)
    pltpu.sync_copy(zb_hbm.at[pl.ds(0, DEN_ROWS)], den_t)
    for t in range(ROWS_PER_TILE // CHUNK + 1):
        sz = min(CHUNK, ROWS_PER_TILE - t * CHUNK)
        if sz <= 0:
            break
        pltpu.sync_copy(rows.at[pl.ds(0, sz)],
                        num_acc.at[pl.ds(r0 + t * CHUNK, sz)])

    pltpu.sync_copy(as_hbm, as_t)
    pltpu.sync_copy(ad_hbm, ad_t)

    plsc.subcore_barrier()

    ebase = wid * EPW
    lane = lax.iota(_i32, L)

    def compute_scatter(j, jj, buf):
        ws = []
        ds16 = []
        for k in range(CHUNK // L):
            s16 = src_idx[jj, pl.ds(k * L, L)]
            d16 = dst_idx[jj, pl.ds(k * L, L)]
            e = plsc.load_gather(as_t, [s16]) + plsc.load_gather(ad_t, [d16])
            e = jnp.where(e >= 0.0, e, e * jnp.float32(0.2))
            w = jnp.exp(e)
            gid = ebase + j * CHUNK + k * L + lane
            w = jnp.where(gid < E, w, jnp.float32(0.0))
            ws.append(w)
            ds16.append(d16)

        for k in range(CHUNK // L):
            for t in range(L):
                i = k * L + t
                sel = lane == t
                m = jnp.sum(jnp.where(sel, ws[k], jnp.float32(0.0)))
                di = jnp.sum(jnp.where(sel, ds16[k], 0))
                for g in range(D // L):
                    sl = pl.ds(g * L, L)
                    buf[i, sl] = buf[i, sl] * m
                dr = lax.shift_right_logical(di, 7)
                dca = lax.bitwise_and(di, 112)
                tin = lax.bitwise_and(di, 15)
                dsl = pl.ds(dca, L)
                den_t[dr, dsl] = den_t[dr, dsl] + jnp.where(
                    lane == tin, m, jnp.float32(0.0))

        pltpu.sync_copy(buf, num_acc.at[dst_idx.at[jj]], add=True)

    def gather(jj, buf, sm):
        return pltpu.async_copy(h_hbm.at[src_idx.at[jj]], buf, sm)

    def block_body(b, carry):
        pltpu.sync_copy(src_hbm.at[wid, pl.ds(b * IDXBLK, IDXBLK)], src_idx)
        pltpu.sync_copy(dst_hbm.at[wid, pl.ds(b * IDXBLK, IDXBLK)], dst_idx)

        gather(0, rows, sem)

        def wait_rows(buf, sm):
            pltpu.make_async_copy(h_hbm.at[src_idx.at[0]], buf, sm).wait()

        def pair_body(jo, c):
            jb = b * IDXBLK
            gather(2 * jo + 1, rows2, sem2)
            wait_rows(rows, sem)
            compute_scatter(jb + 2 * jo, 2 * jo, rows)
            gather(lax.rem(2 * jo + 2, IDXBLK), rows, sem)
            wait_rows(rows2, sem2)
            compute_scatter(jb + 2 * jo + 1, 2 * jo + 1, rows2)
            return c

        lax.fori_loop(0, IDXBLK // 2, pair_body, carry)
        wait_rows(rows, sem)
        return carry

    lax.fori_loop(0, NBLK, block_body, 0)

    pltpu.sync_copy(den_t, den_out.at[wid])

    plsc.subcore_barrier()

    for t in range(ROWS_PER_TILE // CHUNK + 1):
        sz = min(CHUNK, ROWS_PER_TILE - t * CHUNK)
        if sz <= 0:
            break
        r = r0 + t * CHUNK
        pltpu.sync_copy(num_acc.at[pl.ds(r, sz)], rows.at[pl.ds(0, sz)])
        pltpu.sync_copy(rows.at[pl.ds(0, sz)],
                        num_out.at[cid, pl.ds(r, sz)])


_edge_kernel = functools.partial(
    pl.kernel,
    out_type=[
        jax.ShapeDtypeStruct((NC, N_ACC, D), _f32),
        jax.ShapeDtypeStruct((NC * NS, DEN_ROWS, D), _f32),
    ],
    mesh=plsc.VectorSubcoreMesh(core_axis_name="c", subcore_axis_name="s",
                                num_cores=NC, num_subcores=NS),
    compiler_params=pltpu.CompilerParams(needs_layout_passes=False),
    scratch_types=[
        pltpu.VMEM((IDXBLK, CHUNK), _i32),
        pltpu.VMEM((IDXBLK, CHUNK), _i32),
        pltpu.VMEM((N,), _f32),
        pltpu.VMEM((N,), _f32),
        pltpu.VMEM((CHUNK, D), _f32),
        pltpu.VMEM((CHUNK, D), _f32),
        pltpu.VMEM((DEN_ROWS, D), _f32),
        pltpu.SemaphoreType.DMA,
        pltpu.SemaphoreType.DMA,
        pltpu.VMEM_SHARED((N_ACC, D), _f32),
    ],
)(_edge_body)


BN = 400
GRID = N // BN


def _mm_body(x_ref, w_ref, a_ref, h_ref, asad_ref):
    h = jnp.dot(x_ref[...], w_ref[...], preferred_element_type=_f32)
    h_ref[...] = h
    asad_ref[...] = jnp.dot(h, a_ref[...], preferred_element_type=_f32)


def _matmul(x, w, a):
    return pl.pallas_call(
        _mm_body,
        grid=(GRID,),
        in_specs=[
            pl.BlockSpec((BN, D), lambda i: (i, 0)),
            pl.BlockSpec((D, D), lambda i: (0, 0)),
            pl.BlockSpec((D, D), lambda i: (0, 0)),
        ],
        out_specs=[
            pl.BlockSpec((BN, D), lambda i: (i, 0)),
            pl.BlockSpec((BN, D), lambda i: (i, 0)),
        ],
        out_shape=[
            jax.ShapeDtypeStruct((N, D), _f32),
            jax.ShapeDtypeStruct((N, D), _f32),
        ],
    )(x, w, a)


def _combine_mm_body(num_ref, den_ref, b_ref, w_ref, a_ref, h_ref, asad_ref):
    x2 = jnp.maximum(
        num_ref[...] / (den_ref[...] + jnp.float32(1e-16)) + b_ref[...], 0.0)
    h = jnp.dot(x2, w_ref[...], preferred_element_type=_f32)
    h_ref[...] = h
    asad_ref[...] = jnp.dot(h, a_ref[...], preferred_element_type=_f32)


def _combine_mm(num, den, b, w, a):
    return pl.pallas_call(
        _combine_mm_body,
        grid=(GRID,),
        in_specs=[
            pl.BlockSpec((BN, D), lambda i: (i, 0)),
            pl.BlockSpec((BN, 1), lambda i: (i, 0)),
            pl.BlockSpec((1, D), lambda i: (0, 0)),
            pl.BlockSpec((D, D), lambda i: (0, 0)),
            pl.BlockSpec((D, D), lambda i: (0, 0)),
        ],
        out_specs=[
            pl.BlockSpec((BN, D), lambda i: (i, 0)),
            pl.BlockSpec((BN, D), lambda i: (i, 0)),
        ],
        out_shape=[
            jax.ShapeDtypeStruct((N, D), _f32),
            jax.ShapeDtypeStruct((N, D), _f32),
        ],
    )(num, den, b, w, a)


def _combine_body(num_ref, den_ref, b_ref, out_ref):
    out_ref[...] = jnp.maximum(
        num_ref[...] / (den_ref[...] + jnp.float32(1e-16)) + b_ref[...], 0.0)


def _combine(num, den, b):
    return pl.pallas_call(
        _combine_body,
        grid=(GRID,),
        in_specs=[
            pl.BlockSpec((BN, D), lambda i: (i, 0)),
            pl.BlockSpec((BN, 1), lambda i: (i, 0)),
            pl.BlockSpec((1, D), lambda i: (0, 0)),
        ],
        out_specs=pl.BlockSpec((BN, D), lambda i: (i, 0)),
        out_shape=jax.ShapeDtypeStruct((N, D), _f32),
    )(num, den, b)


def kernel(input_x, edge_index, W1, a_src1, a_dst1, b1, W2, a_src2, a_dst2, b2):
    ei = jnp.pad(edge_index, ((0, 0), (0, EP - E)))
    src_r = ei[0].reshape(NC * NS, NCHUNK, CHUNK)
    dst_r = ei[1].reshape(NC * NS, NCHUNK, CHUNK)

    zb = jnp.zeros((N_ACC, D), _f32)

    A1 = jnp.zeros((D, D), _f32).at[:, 0].set(a_src1).at[:, 1].set(a_dst1)
    A2 = jnp.zeros((D, D), _f32).at[:, 0].set(a_src2).at[:, 1].set(a_dst2)

    def _layer_edges(h, as_, ad_):
        nump, denp = _edge_kernel(src_r, dst_r, h, as_, ad_, zb)
        num = (nump[0] + nump[1])[:N]
        den = denp.reshape(NC * NS, DEN_ROWS * D).sum(axis=0)[:N]
        return num, den.reshape(N, 1)

    h1, asad1 = _matmul(input_x, W1, A1)
    num1, den1 = _layer_edges(h1, asad1[:, 0], asad1[:, 1])
    h2, asad2 = _combine_mm(num1, den1, b1.reshape(1, D), W2, A2)
    num2, den2 = _layer_edges(h2, asad2[:, 0], asad2[:, 1])
    return _combine(num2, den2, b2.reshape(1, D))
